# baseline (speedup 1.0000x reference)
import jax
import jax.numpy as jnp
from jax.experimental import pallas as pl
from jax.experimental.pallas import tpu as pltpu

BATCH, SEQ, MODEL_DIM = 4, 4096, 1024
ROWS = BATCH * SEQ
G0 = ROWS // 128
CG = 8
NCHUNK = G0 // CG
NSLOT = 3


def _body(m_ref, s_hbm, e_ref, o_hbm, *rest):
    bufs = rest[:NSLOT]
    sin = rest[NSLOT:2 * NSLOT]
    sout = rest[2 * NSLOT:3 * NSLOT]

    def start_in(c, slot):
        pltpu.make_async_copy(s_hbm.at[pl.ds(c * CG, CG)],
                              bufs[slot], sin[slot]).start()

    def wait_in(slot):
        pltpu.make_async_copy(s_hbm.at[pl.ds(0, CG)],
                              bufs[slot], sin[slot]).wait()

    def start_out(c, slot):
        pltpu.make_async_copy(bufs[slot],
                              o_hbm.at[pl.ds(c * CG, CG)], sout[slot]).start()

    def wait_out(slot):
        pltpu.make_async_copy(bufs[slot],
                              o_hbm.at[pl.ds(0, CG)], sout[slot]).wait()

    for p in range(NSLOT):
        start_in(p, p)

    e = e_ref[...]

    def chunk_body(g, _):
        for b in range(NSLOT):
            c = g * NSLOT + b

            @pl.when(c < NCHUNK)
            def _(b=b, c=c):
                wait_in(b)
                mt = m_ref[pl.ds(c * CG, CG), :].T
                for j in range(CG):
                    mj = mt[:, j:j + 1]
                    bufs[b][j] = jnp.where(mj, e, bufs[b][j])
                start_out(c, b)

                @pl.when(c + NSLOT < NCHUNK)
                def _():
                    wait_out(b)
                    start_in(c + NSLOT, b)
        return 0

    jax.lax.fori_loop(0, (NCHUNK + NSLOT - 1) // NSLOT, chunk_body, 0)

    for s in range(NSLOT):
        wait_out(s)


def kernel(seqs, temporal_mask, temporal_mask_embed):
    mask2d = temporal_mask.reshape(G0, 128)
    seqs3 = seqs.reshape(G0, 128, MODEL_DIM)
    embed2d = temporal_mask_embed.reshape(1, MODEL_DIM)

    out = pl.pallas_call(
        _body,
        in_specs=[
            pl.BlockSpec(memory_space=pltpu.VMEM),
            pl.BlockSpec(memory_space=pl.ANY),
            pl.BlockSpec(memory_space=pltpu.VMEM),
        ],
        out_specs=pl.BlockSpec(memory_space=pl.ANY),
        out_shape=jax.ShapeDtypeStruct((G0, 128, MODEL_DIM), seqs.dtype),
        scratch_shapes=(
            [pltpu.VMEM((CG, 128, MODEL_DIM), jnp.float32)] * NSLOT
            + [pltpu.SemaphoreType.DMA] * (2 * NSLOT)
        ),
    )(mask2d, seqs3, embed2d)
    return (out.reshape(BATCH, SEQ, MODEL_DIM), temporal_mask)
